# baseline (speedup 1.0000x reference)
import functools

import jax
import jax.numpy as jnp
from jax import lax
from jax.experimental import pallas as pl
from jax.experimental.pallas import tpu as pltpu
from jax.experimental.pallas import tpu_sc as plsc

VOCAB = 1000000
D = 64
BATCH = 4096
HIST = 200
NW = 32
BW = BATCH // NW

_mesh = plsc.VectorSubcoreMesh(core_axis_name="c", subcore_axis_name="s")


@functools.partial(
    pl.kernel,
    mesh=_mesh,
    out_type=jax.ShapeDtypeStruct((HIST, 8, NW, 8, 128), jnp.float32),
    scratch_types=[
        pltpu.VMEM((HIST, BW), jnp.int32),
        pltpu.VMEM((BW, D), jnp.float32),
        pltpu.VMEM((BW, D), jnp.float32),
        pltpu.VMEM((8, 8, 128), jnp.float32),
        pltpu.VMEM((8, 8, 128), jnp.float32),
        pltpu.SemaphoreType.DMA,
        pltpu.SemaphoreType.DMA,
        pltpu.SemaphoreType.DMA,
        pltpu.SemaphoreType.DMA,
    ],
    compiler_params=pltpu.CompilerParams(
        use_tc_tiling_on_sc=False, needs_layout_passes=False),
)
def _embed_gather(idx_hbm, table_hbm, out_hbm, idx_all, rows0, rows1,
                  tr0, tr1, sem_g0, sem_g1, sem_s0, sem_s1):
    wid = lax.axis_index("s") * 2 + lax.axis_index("c")
    b0 = wid * BW

    pltpu.sync_copy(idx_hbm.at[:, pl.ds(b0, BW)], idx_all)

    lane = lax.iota(jnp.int32, 16)
    rowv = [lane + 16 * bg for bg in range(8)]

    def fire_gather(h, rows, sem):
        return pltpu.async_copy(table_hbm.at[idx_all.at[h]], rows, sem)

    def wait_gather(h, rows, sem):
        pltpu.make_async_copy(table_hbm.at[idx_all.at[h]], rows, sem).wait()

    def transpose(rows, tr):
        @plsc.parallel_loop(0, D, unroll=8)
        def _(d):
            di = d // 8
            dj = d % 8
            cold = jnp.full((16,), d, jnp.int32)
            for bg in range(8):
                v = plsc.load_gather(rows, [rowv[bg], cold])
                tr[di, dj, pl.ds(16 * bg, 16)] = v

    def fire_store(h, tr, sem):
        return pltpu.async_copy(tr, out_hbm.at[h, :, wid], sem)

    def wait_store(h, tr, sem):
        pltpu.make_async_copy(tr, out_hbm.at[h, :, wid], sem).wait()

    fire_gather(0, rows0, sem_g0)
    fire_gather(1, rows1, sem_g1)

    def pair(m, first, last):
        h0 = 2 * m
        h1 = h0 + 1
        wait_gather(h0, rows0, sem_g0)
        if not first:
            wait_store(h0 - 2, tr0, sem_s0)
        transpose(rows0, tr0)
        if not last:
            fire_gather(h0 + 2, rows0, sem_g0)
        fire_store(h0, tr0, sem_s0)
        wait_gather(h1, rows1, sem_g1)
        if not first:
            wait_store(h1 - 2, tr1, sem_s1)
        transpose(rows1, tr1)
        if not last:
            fire_gather(h1 + 2, rows1, sem_g1)
        fire_store(h1, tr1, sem_s1)
        if last:
            wait_store(h0, tr0, sem_s0)
            wait_store(h1, tr1, sem_s1)

    pair(0, True, False)

    def body(m, carry):
        pair(m, False, False)
        return carry

    lax.fori_loop(1, HIST // 2 - 1, body, 0)
    pair(HIST // 2 - 1, False, True)


def kernel(idx_texts, table):
    idx_t = jnp.transpose(idx_texts).astype(jnp.int32)
    out5 = _embed_gather(idx_t, table)
    return jnp.transpose(out5, (2, 4, 0, 1, 3)).reshape(BATCH, HIST, D)


# speedup vs baseline: 1.0203x; 1.0203x over previous
import functools

import jax
import jax.numpy as jnp
from jax import lax
from jax.experimental import pallas as pl
from jax.experimental.pallas import tpu as pltpu
from jax.experimental.pallas import tpu_sc as plsc

VOCAB = 1000000
D = 64
BATCH = 4096
HIST = 200
NW = 32
BW = BATCH // NW
NBUF = 4

_mesh = plsc.VectorSubcoreMesh(core_axis_name="c", subcore_axis_name="s")

_scratch = (
    [pltpu.VMEM((HIST, BW), jnp.int32)]
    + [pltpu.VMEM((BW, D), jnp.float32) for _ in range(NBUF)]
    + [pltpu.VMEM((8, 8, 128), jnp.float32) for _ in range(NBUF)]
    + [pltpu.SemaphoreType.DMA for _ in range(2 * NBUF)]
)


@functools.partial(
    pl.kernel,
    mesh=_mesh,
    out_type=jax.ShapeDtypeStruct((HIST, 8, NW, 8, 128), jnp.float32),
    scratch_types=_scratch,
    compiler_params=pltpu.CompilerParams(
        use_tc_tiling_on_sc=False, needs_layout_passes=False),
)
def _embed_gather(idx_hbm, table_hbm, out_hbm, idx_all, *bufs):
    rows = list(bufs[0:NBUF])
    trs = list(bufs[NBUF:2 * NBUF])
    sem_g = list(bufs[2 * NBUF:3 * NBUF])
    sem_s = list(bufs[3 * NBUF:4 * NBUF])
    wid = lax.axis_index("s") * 2 + lax.axis_index("c")
    b0 = wid * BW

    pltpu.sync_copy(idx_hbm.at[:, pl.ds(b0, BW)], idx_all)

    lane = lax.iota(jnp.int32, 16)
    rowv = [lane + 16 * bg for bg in range(8)]

    def fire_gather(h, i):
        pltpu.async_copy(table_hbm.at[idx_all.at[h]], rows[i], sem_g[i])

    def wait_gather(h, i):
        pltpu.make_async_copy(
            table_hbm.at[idx_all.at[h]], rows[i], sem_g[i]).wait()

    def transpose(r, tr):
        @plsc.parallel_loop(0, D)
        def _(d):
            di = d // 8
            dj = d % 8
            cold = jnp.full((16,), d, jnp.int32)
            for bg in range(8):
                v = plsc.load_gather(r, [rowv[bg], cold])
                tr[di, dj, pl.ds(16 * bg, 16)] = v

    def fire_store(h, i):
        pltpu.async_copy(trs[i], out_hbm.at[h, :, wid], sem_s[i])

    def wait_store(h, i):
        pltpu.make_async_copy(trs[i], out_hbm.at[h, :, wid], sem_s[i]).wait()

    def step(h, i, first, last):
        wait_gather(h, i)
        if not first:
            wait_store(h - NBUF, i)
        transpose(rows[i], trs[i])
        if not last:
            fire_gather(h + NBUF, i)
        fire_store(h, i)

    for i in range(NBUF):
        fire_gather(i, i)

    def quad(q, first, last):
        h0 = NBUF * q
        for i in range(NBUF):
            step(h0 + i, i, first, last)
        if last:
            for i in range(NBUF):
                wait_store(h0 + i, i)

    quad(0, True, False)

    def body(q, carry):
        quad(q, False, False)
        return carry

    lax.fori_loop(1, HIST // NBUF - 1, body, 0)
    quad(HIST // NBUF - 1, False, True)


def kernel(idx_texts, table):
    idx_t = jnp.transpose(idx_texts).astype(jnp.int32)
    out5 = _embed_gather(idx_t, table)
    return jnp.transpose(out5, (2, 4, 0, 1, 3)).reshape(BATCH, HIST, D)
